# baseline (speedup 1.0000x reference)
import functools
import math

import jax
import jax.numpy as jnp
from jax.experimental import pallas as pl
from jax.experimental.pallas import tpu as pltpu

N_HEADS = 16
HEAD_DIM = 128


def _matmul_bias_cast_kernel(x_ref, w_ref, b_ref, o_ref, xb_ref):
    @pl.when(pl.program_id(0) == 0)
    def _():
        xb_ref[...] = x_ref[...].astype(jnp.bfloat16)
    acc = jax.lax.dot_general(
        xb_ref[...],
        w_ref[...].astype(jnp.bfloat16),
        (((1,), (1,)), ((), ())),
        preferred_element_type=jnp.float32,
    ) + b_ref[...]
    o_ref[...] = acc.astype(o_ref.dtype)


def _matmul_bias_kernel(x_ref, w_ref, b_ref, o_ref):
    acc = jax.lax.dot_general(
        x_ref[...],
        w_ref[...].astype(jnp.bfloat16),
        (((1,), (1,)), ((), ())),
        preferred_element_type=jnp.float32,
    ) + b_ref[...]
    o_ref[...] = acc.astype(o_ref.dtype)


def _matmul_bias(x, w, b, bn, out_dtype):
    t, k = x.shape
    n = w.shape[0]
    grid = (n // bn,)
    needs_cast = x.dtype == jnp.float32
    return pl.pallas_call(
        _matmul_bias_cast_kernel if needs_cast else _matmul_bias_kernel,
        grid=grid,
        in_specs=[
            pl.BlockSpec((t, k), lambda j: (0, 0)),
            pl.BlockSpec((bn, k), lambda j: (j, 0)),
            pl.BlockSpec((1, bn), lambda j: (0, j)),
        ],
        out_specs=pl.BlockSpec((t, bn), lambda j: (0, j)),
        out_shape=jax.ShapeDtypeStruct((t, n), out_dtype),
        scratch_shapes=(
            [pltpu.VMEM((t, k), jnp.bfloat16)] if needs_cast else []
        ),
        compiler_params=pltpu.CompilerParams(
            dimension_semantics=("parallel",),
        ),
    )(x, w, b.reshape(1, n))


def _flash_head_kernel(q_ref, k_ref, v_ref, o_ref, vaug_ref, *, bq, bk, scale):
    t = q_ref.shape[0]
    hs = HEAD_DIM
    nq = t // bq
    log2e = 1.4426950408889634

    ones_h = jnp.ones((hs, 128), jnp.bfloat16)
    qb = q_ref[...]
    qn = jax.lax.dot_general(
        qb * qb, ones_h, (((1,), (0,)), ((), ())),
        preferred_element_type=jnp.float32,
    )
    kb = k_ref[...]
    kn = jax.lax.dot_general(
        kb * kb, ones_h, (((1,), (0,)), ((), ())),
        preferred_element_type=jnp.float32,
    )
    m_r = jnp.sqrt(jnp.max(qn)) * jnp.sqrt(jnp.max(kn)) * (scale * 1.05)
    c1 = jnp.float32(scale * log2e)
    c2 = m_r * jnp.float32(log2e)

    vaug_ref[:, :hs] = v_ref[...]
    vaug_ref[:, hs:] = jnp.ones((t, hs), jnp.bfloat16)

    rows = jax.lax.broadcasted_iota(jnp.int32, (bq, bk), 0)
    cols = jax.lax.broadcasted_iota(jnp.int32, (bq, bk), 1)
    diag_mask = rows >= cols

    for ib in range(nq):
        q = qb[ib * bq:(ib + 1) * bq, :]
        acc = None
        for j in range(ib + 1):
            kc = kb[j * bk:(j + 1) * bk, :]
            s = jax.lax.dot_general(
                q, kc, (((1,), (1,)), ((), ())),
                preferred_element_type=jnp.float32,
            )
            p = jnp.exp2(s * c1 - c2)
            if j == ib:
                p = jnp.where(diag_mask, p, 0.0)
            vc = vaug_ref[j * bk:(j + 1) * bk, :]
            pv = jax.lax.dot_general(
                p.astype(jnp.bfloat16), vc, (((1,), (0,)), ((), ())),
                preferred_element_type=jnp.float32,
            )
            acc = pv if acc is None else acc + pv
        o_ref[ib * bq:(ib + 1) * bq, :] = (
            acc[:, :hs] / acc[:, hs:]).astype(o_ref.dtype)


def _flash_attention(qkv, t, c, bq, bk):
    h = N_HEADS
    hs = HEAD_DIM
    hb = c // hs
    scale = 1.0 / math.sqrt(hs)
    kern = functools.partial(_flash_head_kernel, bq=bq, bk=bk, scale=scale)
    return pl.pallas_call(
        kern,
        grid=(h,),
        in_specs=[
            pl.BlockSpec((t, hs), lambda hh: (0, hh)),
            pl.BlockSpec((t, hs), lambda hh: (0, hb + hh)),
            pl.BlockSpec((t, hs), lambda hh: (0, 2 * hb + hh)),
        ],
        out_specs=pl.BlockSpec((t, hs), lambda hh: (0, hh)),
        out_shape=jax.ShapeDtypeStruct((t, c), jnp.bfloat16),
        scratch_shapes=[
            pltpu.VMEM((t, 2 * hs), jnp.bfloat16),
        ],
        compiler_params=pltpu.CompilerParams(
            dimension_semantics=("parallel",),
        ),
    )(qkv, qkv, qkv)


@jax.jit
def _attention_impl(x, W_attn, b_attn, W_proj, b_proj):
    b, t, c = x.shape
    x2 = x.reshape(t, c)
    qkv = _matmul_bias(x2, W_attn, b_attn, bn=768, out_dtype=jnp.bfloat16)
    y = _flash_attention(qkv, t, c, bq=512, bk=512)
    out = _matmul_bias(y, W_proj, b_proj, bn=1024, out_dtype=jnp.float32)
    return out.reshape(b, t, c)


def kernel(x, W_attn, b_attn, W_proj, b_proj):
    return _attention_impl(x, W_attn, b_attn, W_proj, b_proj)


# speedup vs baseline: 1.0030x; 1.0030x over previous
import functools
import math

import jax
import jax.numpy as jnp
from jax.experimental import pallas as pl
from jax.experimental.pallas import tpu as pltpu

N_HEADS = 16
HEAD_DIM = 128


def _matmul_bias_cast_kernel(x_ref, w_ref, b_ref, o_ref, xb_ref):
    @pl.when(pl.program_id(0) == 0)
    def _():
        xb_ref[...] = x_ref[...].astype(jnp.bfloat16)
    acc = jax.lax.dot_general(
        xb_ref[...],
        w_ref[...].astype(jnp.bfloat16),
        (((1,), (1,)), ((), ())),
        preferred_element_type=jnp.float32,
    ) + b_ref[...]
    o_ref[...] = acc.astype(o_ref.dtype)


def _matmul_bias_kernel(x_ref, w_ref, b_ref, o_ref):
    acc = jax.lax.dot_general(
        x_ref[...],
        w_ref[...].astype(jnp.bfloat16),
        (((1,), (1,)), ((), ())),
        preferred_element_type=jnp.float32,
    ) + b_ref[...]
    o_ref[...] = acc.astype(o_ref.dtype)


def _matmul_bias(x, w, b, bn, out_dtype):
    t, k = x.shape
    n = w.shape[0]
    grid = (n // bn,)
    needs_cast = x.dtype == jnp.float32
    return pl.pallas_call(
        _matmul_bias_cast_kernel if needs_cast else _matmul_bias_kernel,
        grid=grid,
        in_specs=[
            pl.BlockSpec((t, k), lambda j: (0, 0)),
            pl.BlockSpec((bn, k), lambda j: (j, 0)),
            pl.BlockSpec((1, bn), lambda j: (0, j)),
        ],
        out_specs=pl.BlockSpec((t, bn), lambda j: (0, j)),
        out_shape=jax.ShapeDtypeStruct((t, n), out_dtype),
        scratch_shapes=(
            [pltpu.VMEM((t, k), jnp.bfloat16)] if needs_cast else []
        ),
        compiler_params=pltpu.CompilerParams(
            dimension_semantics=("parallel",),
        ),
    )(x, w, b.reshape(1, n))


def _flash_head_kernel(q_ref, k_ref, v_ref, o_ref, vaug_ref, *, bq, bk, scale):
    t = q_ref.shape[0]
    hs = HEAD_DIM
    nq = t // bq
    log2e = 1.4426950408889634

    ones_h = jnp.ones((hs, 128), jnp.bfloat16)
    qb = q_ref[...]
    qn = jax.lax.dot_general(
        qb * qb, ones_h, (((1,), (0,)), ((), ())),
        preferred_element_type=jnp.float32,
    )
    kb = k_ref[...]
    kn = jax.lax.dot_general(
        kb * kb, ones_h, (((1,), (0,)), ((), ())),
        preferred_element_type=jnp.float32,
    )
    m_r = jnp.sqrt(jnp.max(qn)) * jnp.sqrt(jnp.max(kn)) * (scale * 1.05)
    c1 = jnp.float32(scale * log2e)
    c2 = m_r * jnp.float32(log2e)

    vaug_ref[:, :hs] = v_ref[...]
    vaug_ref[:, hs:] = jnp.ones((t, hs), jnp.bfloat16)

    rows = jax.lax.broadcasted_iota(jnp.int32, (bq, bk), 0)
    cols = jax.lax.broadcasted_iota(jnp.int32, (bq, bk), 1)
    diag_mask = rows >= cols

    for ib in range(nq):
        q = qb[ib * bq:(ib + 1) * bq, :]
        acc = None
        for j in range(ib + 1):
            kc = kb[j * bk:(j + 1) * bk, :]
            s = jax.lax.dot_general(
                q, kc, (((1,), (1,)), ((), ())),
                preferred_element_type=jnp.float32,
            )
            p = jnp.exp2(s * c1 - c2)
            if j == ib:
                p = jnp.where(diag_mask, p, 0.0)
            vc = vaug_ref[j * bk:(j + 1) * bk, :]
            pv = jax.lax.dot_general(
                p.astype(jnp.bfloat16), vc, (((1,), (0,)), ((), ())),
                preferred_element_type=jnp.float32,
            )
            acc = pv if acc is None else acc + pv
        o_ref[ib * bq:(ib + 1) * bq, :] = (
            acc[:, :hs] / acc[:, hs:]).astype(o_ref.dtype)


def _flash_attention(qkv, t, c, bq, bk):
    h = N_HEADS
    hs = HEAD_DIM
    hb = c // hs
    scale = 1.0 / math.sqrt(hs)
    kern = functools.partial(_flash_head_kernel, bq=bq, bk=bk, scale=scale)
    return pl.pallas_call(
        kern,
        grid=(h,),
        in_specs=[
            pl.BlockSpec((t, hs), lambda hh: (0, hh)),
            pl.BlockSpec((t, hs), lambda hh: (0, hb + hh)),
            pl.BlockSpec((t, hs), lambda hh: (0, 2 * hb + hh)),
        ],
        out_specs=pl.BlockSpec((t, hs), lambda hh: (0, hh)),
        out_shape=jax.ShapeDtypeStruct((t, c), jnp.bfloat16),
        scratch_shapes=[
            pltpu.VMEM((t, 2 * hs), jnp.bfloat16),
        ],
        compiler_params=pltpu.CompilerParams(
            dimension_semantics=("parallel",),
        ),
    )(qkv, qkv, qkv)


@jax.jit
def _attention_impl(x, W_attn, b_attn, W_proj, b_proj):
    b, t, c = x.shape
    x2 = x.reshape(t, c)
    qkv = _matmul_bias(x2, W_attn, b_attn, bn=512, out_dtype=jnp.bfloat16)
    y = _flash_attention(qkv, t, c, bq=512, bk=512)
    out = _matmul_bias(y, W_proj, b_proj, bn=1024, out_dtype=jnp.float32)
    return out.reshape(b, t, c)


def kernel(x, W_attn, b_attn, W_proj, b_proj):
    return _attention_impl(x, W_attn, b_attn, W_proj, b_proj)


# speedup vs baseline: 1.0819x; 1.0787x over previous
import functools
import math

import jax
import jax.numpy as jnp
from jax.experimental import pallas as pl
from jax.experimental.pallas import tpu as pltpu

N_HEADS = 16
HEAD_DIM = 128


def _matmul_bias_cast_kernel(x_ref, w_ref, b_ref, o_ref, xb_ref):
    @pl.when(pl.program_id(0) == 0)
    def _():
        xb_ref[...] = x_ref[...].astype(jnp.bfloat16)
    acc = jax.lax.dot_general(
        xb_ref[...],
        w_ref[...].astype(jnp.bfloat16),
        (((1,), (1,)), ((), ())),
        preferred_element_type=jnp.float32,
    ) + b_ref[...]
    o_ref[...] = acc.astype(o_ref.dtype)


def _matmul_bias_kernel(x_ref, w_ref, b_ref, o_ref):
    acc = jax.lax.dot_general(
        x_ref[...],
        w_ref[...].astype(jnp.bfloat16),
        (((1,), (1,)), ((), ())),
        preferred_element_type=jnp.float32,
    ) + b_ref[...]
    o_ref[...] = acc.astype(o_ref.dtype)


def _matmul_bias(x, w, b, bn, out_dtype):
    t, k = x.shape
    n = w.shape[0]
    grid = (n // bn,)
    needs_cast = x.dtype == jnp.float32
    return pl.pallas_call(
        _matmul_bias_cast_kernel if needs_cast else _matmul_bias_kernel,
        grid=grid,
        in_specs=[
            pl.BlockSpec((t, k), lambda j: (0, 0)),
            pl.BlockSpec((bn, k), lambda j: (j, 0)),
            pl.BlockSpec((1, bn), lambda j: (0, j)),
        ],
        out_specs=pl.BlockSpec((t, bn), lambda j: (0, j)),
        out_shape=jax.ShapeDtypeStruct((t, n), out_dtype),
        scratch_shapes=(
            [pltpu.VMEM((t, k), jnp.bfloat16)] if needs_cast else []
        ),
        compiler_params=pltpu.CompilerParams(
            dimension_semantics=("parallel",),
        ),
    )(x, w, b.reshape(1, n))


def _flash_head_kernel(q_ref, k_ref, v_ref, o_ref, vaug_ref, *, bq, bk, scale):
    t = q_ref.shape[0]
    hs = HEAD_DIM
    nq = t // bq
    log2e = 1.4426950408889634

    ones_h = jnp.ones((hs, 128), jnp.bfloat16)
    qb = q_ref[...]
    qn = jax.lax.dot_general(
        qb * qb, ones_h, (((1,), (0,)), ((), ())),
        preferred_element_type=jnp.float32,
    )
    kb = k_ref[...]
    kn = jax.lax.dot_general(
        kb * kb, ones_h, (((1,), (0,)), ((), ())),
        preferred_element_type=jnp.float32,
    )
    m_r = jnp.sqrt(jnp.max(qn)) * jnp.sqrt(jnp.max(kn)) * (scale * 1.05)
    c1 = jnp.float32(scale * log2e)
    c2 = m_r * jnp.float32(log2e)

    vaug_ref[:, :hs] = v_ref[...]
    vaug_ref[:, hs:] = jnp.ones((t, hs), jnp.bfloat16)

    rows = jax.lax.broadcasted_iota(jnp.int32, (bq, bk), 0)
    cols = jax.lax.broadcasted_iota(jnp.int32, (bq, bk), 1)
    diag_mask = rows >= cols

    for ib in range(nq):
        q = qb[ib * bq:(ib + 1) * bq, :]
        acc = None
        for j in range(ib + 1):
            kc = kb[j * bk:(j + 1) * bk, :]
            s = jax.lax.dot_general(
                q, kc, (((1,), (1,)), ((), ())),
                preferred_element_type=jnp.float32,
            )
            p = jnp.exp2(s * c1 - c2)
            if j == ib:
                p = jnp.where(diag_mask, p, 0.0)
            vc = vaug_ref[j * bk:(j + 1) * bk, :]
            pv = jax.lax.dot_general(
                p.astype(jnp.bfloat16), vc, (((1,), (0,)), ((), ())),
                preferred_element_type=jnp.float32,
            )
            acc = pv if acc is None else acc + pv
        o_ref[ib * bq:(ib + 1) * bq, :] = (
            acc[:, :hs] / acc[:, hs:]).astype(o_ref.dtype)


def _flash_attention(qkv, t, c, bq, bk):
    h = N_HEADS
    hs = HEAD_DIM
    hb = c // hs
    scale = 1.0 / math.sqrt(hs)
    kern = functools.partial(_flash_head_kernel, bq=bq, bk=bk, scale=scale)
    return pl.pallas_call(
        kern,
        grid=(h,),
        in_specs=[
            pl.BlockSpec((t, hs), lambda hh: (0, hh)),
            pl.BlockSpec((t, hs), lambda hh: (0, hb + hh)),
            pl.BlockSpec((t, hs), lambda hh: (0, 2 * hb + hh)),
        ],
        out_specs=pl.BlockSpec((t, hs), lambda hh: (0, hh)),
        out_shape=jax.ShapeDtypeStruct((t, c), jnp.bfloat16),
        scratch_shapes=[
            pltpu.VMEM((t, 2 * hs), jnp.bfloat16),
        ],
        compiler_params=pltpu.CompilerParams(
            dimension_semantics=("parallel",),
        ),
    )(qkv, qkv, qkv)


@jax.jit
def _attention_impl(x, W_attn, b_attn, W_proj, b_proj):
    b, t, c = x.shape
    x2 = x.reshape(t, c)
    qkv = _matmul_bias(x2, W_attn, b_attn, bn=512, out_dtype=jnp.bfloat16)
    y = _flash_attention(qkv, t, c, bq=256, bk=256)
    out = _matmul_bias(y, W_proj, b_proj, bn=512, out_dtype=jnp.float32)
    return out.reshape(b, t, c)


def kernel(x, W_attn, b_attn, W_proj, b_proj):
    return _attention_impl(x, W_attn, b_attn, W_proj, b_proj)
